# baseline (speedup 1.0000x reference)
import functools

import jax
import jax.numpy as jnp
from jax import lax
from jax.experimental import pallas as pl
from jax.experimental.pallas import tpu as pltpu
from jax.experimental.pallas import tpu_sc as plsc

DIM = 64
B = 16384
NEG = 20
J = NEG + 1
NC = 2
NS = 16
NW = NC * NS
BPW = B // NW
QCH = 128
QN = BPW // QCH
LANES = 16


@functools.partial(
    pl.kernel,
    mesh=plsc.VectorSubcoreMesh(core_axis_name="c", subcore_axis_name="s"),
    compiler_params=pltpu.CompilerParams(use_tc_tiling_on_sc=False,
                                         needs_layout_passes=False),
    out_type=jax.ShapeDtypeStruct((NW, J, BPW), jnp.float32),
    scratch_types=[
        pltpu.VMEM((QN, QCH), jnp.int32),
        pltpu.VMEM((J, QN, QCH), jnp.int32),
        pltpu.VMEM((BPW, DIM), jnp.float32),
        pltpu.VMEM((2, BPW, DIM), jnp.float32),
        pltpu.VMEM((J, BPW), jnp.float32),
        pltpu.SemaphoreType.DMA,
        pltpu.SemaphoreType.DMA,
    ],
)
def _sc_scores(tidx_hbm, cn_hbm, wt_hbm, wc_hbm, out_hbm,
               tidx_v, cidx_v, t_rows, r_buf, scores_v, sem0, sem1):
    wid = lax.axis_index("s") * NC + lax.axis_index("c")

    pltpu.sync_copy(tidx_hbm.at[wid], tidx_v)
    pltpu.sync_copy(cn_hbm.at[:, wid], cidx_v)

    for q in range(QN):
        pltpu.async_copy(wt_hbm.at[tidx_v.at[q]],
                         t_rows.at[pl.ds(q * QCH, QCH)], sem0).wait()

    lane = jnp.arange(LANES, dtype=jnp.int32)
    sems = (sem0, sem1)

    def start_gather(j, b):
        for q in range(QN):
            pltpu.async_copy(wc_hbm.at[cidx_v.at[j, q]],
                             r_buf.at[b, pl.ds(q * QCH, QCH)], sems[b])

    def drain(b):
        pltpu.make_async_copy(wc_hbm.at[pl.ds(0, BPW)],
                              r_buf.at[b], sems[b]).wait()

    def compute(j, b):
        def blk_body(blk, c):
            rows = blk * LANES + lane
            acc = jnp.zeros((LANES,), jnp.float32)
            for d in range(DIM):
                col = jnp.full((LANES,), d, jnp.int32)
                acc = acc + (plsc.load_gather(t_rows, [rows, col])
                             * plsc.load_gather(r_buf.at[b], [rows, col]))
            scores_v[j, pl.ds(blk * LANES, LANES)] = acc
            return c
        lax.fori_loop(0, BPW // LANES, blk_body, 0)

    start_gather(0, 0)

    def j_body(p, carry):
        for b in range(2):
            j = p * 2 + b

            @pl.when(j < J)
            def _():
                drain(b)

                @pl.when(j + 1 < J)
                def _():
                    start_gather(j + 1, 1 - b)

                compute(j, b)
        return carry

    lax.fori_loop(0, (J + 1) // 2, j_body, 0)
    pltpu.sync_copy(scores_v, out_hbm.at[wid])


def _tc_loss_body(s_ref, o_ref):
    s = s_ref[...]
    row = lax.broadcasted_iota(jnp.int32, s.shape, 0)
    x = jnp.where(row % J == 0, -s, s)
    sp = jnp.maximum(x, 0.0) + jnp.log1p(jnp.exp(-jnp.abs(x)))
    o_ref[0, 0] = jnp.sum(sp) * (1.0 / B)


def kernel(target, context, negatives, W_target, W_context):
    tgt = target.astype(jnp.int32)
    cn = jnp.concatenate(
        [context.astype(jnp.int32)[None, :], negatives.astype(jnp.int32).T],
        axis=0)
    tidx = tgt.reshape(NW, QN, QCH)
    cnidx = cn.reshape(J, NW, QN, QCH)

    scores = _sc_scores(tidx, cnidx, W_target, W_context)

    loss = pl.pallas_call(
        _tc_loss_body,
        out_shape=jax.ShapeDtypeStruct((1, 1), jnp.float32),
        out_specs=pl.BlockSpec(memory_space=pltpu.SMEM),
    )(scores.reshape(NW * J, BPW))
    return loss[0, 0]
